# baseline (speedup 1.0000x reference)
import jax
import jax.numpy as jnp
import numpy as np
from jax.experimental import pallas as pl


def _knot_maps():
    M = np.zeros((10, 12), np.float32)
    C = np.zeros((1, 12), np.float32)
    M[0, 0] = -1.0; M[1, 0] = -1.0; C[0, 0] = -10000.0
    M[0, 1] = -1.0; M[1, 1] = -1.0
    M[1, 2] = -1.0
    M[2, 3] = 1.0
    M[2, 4] = 1.0; M[3, 4] = 1.0
    M[2, 5] = 1.0; M[3, 5] = 1.0; C[0, 5] = 10000.0
    M[4, 6] = -1.0; M[5, 6] = -1.0; M[8, 6] = -20000.0
    M[4, 7] = -1.0; M[5, 7] = -1.0
    M[5, 8] = -1.0
    M[6, 9] = 1.0
    M[6, 10] = 1.0; M[7, 10] = 1.0
    M[6, 11] = 1.0; M[7, 11] = 1.0; M[9, 11] = 20000.0
    return jnp.asarray(M), jnp.asarray(C)


def _body(x_ref, t_ref, mask_ref, W1_ref, b1_ref, W2_ref, b2_ref, M_ref, C_ref,
          out_ref):
    x = x_ref[...]
    z = jnp.tanh(x[:, 0:2])
    h = (z @ W1_ref[0:2, :] + t_ref[...] @ W1_ref[2:, :]) + b1_ref[...]
    h = jnp.maximum(h, 0.0)
    p = h @ W2_ref[...] + b2_ref[...]
    d = jax.nn.softplus(p) + 1e-4
    axy = d @ M_ref[...] + C_ref[...]
    ax = axy[:, 0:6]
    ay = axy[:, 6:12]
    qx = x[:, 2:3]
    qx = jnp.clip(qx, ax[:, 0:1] * 0.99, ax[:, 5:6] * 0.99)
    xl = ax[:, 0:5]
    xr = ax[:, 1:6]
    yl = ay[:, 0:5]
    yr = ay[:, 1:6]
    sel = ((qx >= xl) & (qx < xr)).astype(jnp.float32)
    cand = (yr - yl) / (xr - xl) * (qx - xl) + yl
    gy = jnp.sum(cand * sel, axis=1, keepdims=True)
    ym = x * mask_ref[...]
    out_ref[...] = jnp.concatenate([ym[:, 0:2], gy], axis=1)


def kernel(x, t_feat, mask, W1, b1, W2, b2):
    n, p, _ = x.shape
    rows = n * p
    blk = 2048
    grid = rows // blk
    M, C = _knot_maps()
    const = lambda i: (0, 0)
    out = pl.pallas_call(
        _body,
        grid=(grid,),
        in_specs=[
            pl.BlockSpec((blk, 3), lambda i: (i, 0)),
            pl.BlockSpec((blk, 64), lambda i: (i, 0)),
            pl.BlockSpec((1, 3), const),
            pl.BlockSpec((66, 128), const),
            pl.BlockSpec((1, 128), const),
            pl.BlockSpec((128, 10), const),
            pl.BlockSpec((1, 10), const),
            pl.BlockSpec((10, 12), const),
            pl.BlockSpec((1, 12), const),
        ],
        out_specs=pl.BlockSpec((blk, 3), lambda i: (i, 0)),
        out_shape=jax.ShapeDtypeStruct((rows, 3), jnp.float32),
    )(x.reshape(rows, 3), t_feat.reshape(rows, 64), mask.reshape(1, 3),
      W1, b1.reshape(1, 128), W2, b2.reshape(1, 10), M, C)
    return out.reshape(n, p, 3)


# speedup vs baseline: 4.0163x; 4.0163x over previous
import jax
import jax.numpy as jnp
import numpy as np
from jax.experimental import pallas as pl


def _knot_maps():
    M = np.zeros((12, 10), np.float32)
    C = np.zeros((12, 1), np.float32)
    M[0, 0] = -1.0; M[0, 1] = -1.0; C[0, 0] = -10000.0
    M[1, 0] = -1.0; M[1, 1] = -1.0
    M[2, 1] = -1.0
    M[3, 2] = 1.0
    M[4, 2] = 1.0; M[4, 3] = 1.0
    M[5, 2] = 1.0; M[5, 3] = 1.0; C[5, 0] = 10000.0
    M[6, 4] = -1.0; M[6, 5] = -1.0; M[6, 8] = -20000.0
    M[7, 4] = -1.0; M[7, 5] = -1.0
    M[8, 5] = -1.0
    M[9, 6] = 1.0
    M[10, 6] = 1.0; M[10, 7] = 1.0
    M[11, 6] = 1.0; M[11, 7] = 1.0; M[11, 9] = 20000.0
    return jnp.asarray(M), jnp.asarray(C)


def _body(xT_ref, t_ref, mask_ref, W1zT_ref, W1tT_ref, b1_ref, W2T_ref,
          b2_ref, M_ref, C_ref, out_ref):
    xT = xT_ref[...]
    zT = jnp.tanh(xT[0:2, :])
    tT = jnp.transpose(t_ref[...])
    hT = W1zT_ref[...] @ zT + W1tT_ref[...] @ tT + b1_ref[...]
    hT = jnp.maximum(hT, 0.0)
    pT = W2T_ref[...] @ hT + b2_ref[...]
    dT = jax.nn.softplus(pT) + 1e-4
    axy = M_ref[...] @ dT + C_ref[...]
    ax = axy[0:6, :]
    ay = axy[6:12, :]
    qx = jnp.clip(xT[2:3, :], ax[0:1, :] * 0.99, ax[5:6, :] * 0.99)
    xl = ax[0:5, :]
    xr = ax[1:6, :]
    yl = ay[0:5, :]
    yr = ay[1:6, :]
    sel = ((qx >= xl) & (qx < xr)).astype(jnp.float32)
    cand = (yr - yl) / (xr - xl) * (qx - xl) + yl
    gy = jnp.sum(cand * sel, axis=0, keepdims=True)
    out_ref[...] = jnp.concatenate([xT[0:2, :] * mask_ref[0:2, :], gy], axis=0)


def kernel(x, t_feat, mask, W1, b1, W2, b2):
    n, p, _ = x.shape
    rows = n * p
    blk = 2048
    grid = rows // blk
    M, C = _knot_maps()
    xT = x.reshape(rows, 3).T
    const = lambda i: (0, 0)
    outT = pl.pallas_call(
        _body,
        grid=(grid,),
        in_specs=[
            pl.BlockSpec((3, blk), lambda i: (0, i)),
            pl.BlockSpec((blk, 64), lambda i: (i, 0)),
            pl.BlockSpec((3, 1), const),
            pl.BlockSpec((128, 2), const),
            pl.BlockSpec((128, 64), const),
            pl.BlockSpec((128, 1), const),
            pl.BlockSpec((10, 128), const),
            pl.BlockSpec((10, 1), const),
            pl.BlockSpec((12, 10), const),
            pl.BlockSpec((12, 1), const),
        ],
        out_specs=pl.BlockSpec((3, blk), lambda i: (0, i)),
        out_shape=jax.ShapeDtypeStruct((3, rows), jnp.float32),
    )(xT, t_feat.reshape(rows, 64), mask.reshape(3, 1),
      W1[0:2, :].T, W1[2:, :].T, b1.reshape(128, 1),
      W2.T, b2.reshape(10, 1), M, C)
    return outT.T.reshape(n, p, 3)
